# baseline (speedup 1.0000x reference)
import functools

import jax
import jax.numpy as jnp
from jax import lax
from jax.experimental import pallas as pl
from jax.experimental.pallas import tpu as pltpu
from jax.experimental.pallas import tpu_sc as plsc

N = 10000
E = 320000
D = 128
L = 2
T = 2
M = 10240
RPS = M // 16
BLK = 128
NBLK = E // BLK
NW = 32
QB = 80
NBLKP = NW * QB
EPAD = NBLKP * BLK - E
INNER = 10

_mesh = plsc.VectorSubcoreMesh(core_axis_name="c", subcore_axis_name="s")


def _conv_body(y_hbm, src_hbm, dst_hbm, out_hbm, acc, sblk,
               dblk, rows_a, rows_b, sem_a, sem_b, sem_i):
    c = lax.axis_index("c")
    s = lax.axis_index("s")
    wid = c * 16 + s
    base_e = wid * QB * BLK
    HQ = QB // 2
    HE = HQ * BLK

    rows = (rows_a, rows_b)
    sems = (sem_a, sem_b)

    nch = RPS // BLK
    for k in range(nch):
        r0 = s * RPS + k * BLK
        pltpu.sync_copy(y_hbm.at[pl.ds(r0, BLK)], rows_a)
        pltpu.sync_copy(rows_a, acc.at[pl.ds(r0, BLK)])
    plsc.subcore_barrier()

    def blk(b, carry):
        off = base_e + b * BLK
        pltpu.sync_copy(src_hbm.at[pl.ds(off, BLK)], sblk)
        pltpu.sync_copy(dst_hbm.at[pl.ds(off, BLK)], dblk)
        pltpu.async_copy(y_hbm.at[sblk], rows_a, sem_a).wait()
        pltpu.sync_copy(rows_a, acc.at[dblk], add=True)
        return carry

    lax.fori_loop(0, QB, blk, 0)
    plsc.subcore_barrier()

    for k in range(nch):
        r0 = s * RPS + k * BLK
        pltpu.sync_copy(acc.at[pl.ds(r0, BLK)], rows_a)
        pltpu.sync_copy(rows_a, out_hbm.at[c, pl.ds(r0, BLK)])


@jax.jit
def _conv_sc(y, src_blk, dst_blk):
    return pl.kernel(
        _conv_body,
        out_type=jax.ShapeDtypeStruct((2, M, D), jnp.float32),
        mesh=_mesh,
        scratch_types=[
            pltpu.VMEM_SHARED((M, D), jnp.float32),
            pltpu.VMEM((BLK,), jnp.int32),
            pltpu.VMEM((BLK,), jnp.int32),
            pltpu.VMEM((BLK, D), jnp.float32),
            pltpu.VMEM((BLK, D), jnp.float32),
            pltpu.SemaphoreType.DMA,
            pltpu.SemaphoreType.DMA,
            pltpu.SemaphoreType.DMA,
        ],
    )(y, src_blk, dst_blk)


def _deg_body(dst0_hbm, dst1_hbm, out_hbm, acc0, acc1, ones_v, didx, iov):
    c = lax.axis_index("c")
    s = lax.axis_index("s")
    wid = c * 16 + s

    for i in range(BLK):
        ones_v[i] = jnp.ones((16,), jnp.float32)
        iov[i] = jnp.zeros((16,), jnp.float32)
    for k in range(RPS // BLK):
        r0 = s * RPS + k * BLK
        pltpu.sync_copy(iov, acc0.at[pl.ds(r0, BLK)])
        pltpu.sync_copy(iov, acc1.at[pl.ds(r0, BLK)])
    plsc.subcore_barrier()

    q, r = divmod(NBLK, NW)
    nblk = q + jnp.where(wid < r, 1, 0)
    base = wid * q + jnp.minimum(wid, r)

    def blk0(j, carry):
        off = (base + j) * BLK
        pltpu.sync_copy(dst0_hbm.at[pl.ds(off, BLK)], didx)
        pltpu.sync_copy(ones_v, acc0.at[didx], add=True)
        return carry

    def blk1(j, carry):
        off = (base + j) * BLK
        pltpu.sync_copy(dst1_hbm.at[pl.ds(off, BLK)], didx)
        pltpu.sync_copy(ones_v, acc1.at[didx], add=True)
        return carry

    lax.fori_loop(0, nblk, blk0, 0)
    lax.fori_loop(0, nblk, blk1, 0)
    plsc.subcore_barrier()

    for k in range(RPS // BLK):
        r0 = s * RPS + k * BLK
        pltpu.sync_copy(acc0.at[pl.ds(r0, BLK)], iov)
        pltpu.sync_copy(iov, out_hbm.at[c, 0, pl.ds(r0, BLK)])
        pltpu.sync_copy(acc1.at[pl.ds(r0, BLK)], iov)
        pltpu.sync_copy(iov, out_hbm.at[c, 1, pl.ds(r0, BLK)])


@jax.jit
def _deg_sc(dst0, dst1):
    return pl.kernel(
        _deg_body,
        out_type=jax.ShapeDtypeStruct((2, 2, M, 16), jnp.float32),
        mesh=_mesh,
        scratch_types=[
            pltpu.VMEM_SHARED((M, 16), jnp.float32),
            pltpu.VMEM_SHARED((M, 16), jnp.float32),
            pltpu.VMEM((BLK, 16), jnp.float32),
            pltpu.VMEM((BLK,), jnp.int32),
            pltpu.VMEM((BLK, 16), jnp.float32),
        ],
    )(dst0, dst1)


def _dinv_body(da_ref, out_ref):
    d0 = da_ref[0, 0] + da_ref[1, 0]
    out_ref[0] = lax.rsqrt(jnp.sum(d0, axis=1, keepdims=True) + 1.0)
    d1 = da_ref[0, 1] + da_ref[1, 1]
    out_ref[1] = lax.rsqrt(jnp.sum(d1, axis=1, keepdims=True) + 1.0)


@jax.jit
def _dinv_tc(degacc):
    return pl.pallas_call(
        _dinv_body,
        out_shape=jax.ShapeDtypeStruct((2, M, 1), jnp.float32),
    )(degacc)


def _gru_body(x_ref, h_ref, wih_ref, whh_ref, bih_ref, bhh_ref, out_ref):
    x = x_ref[...]
    h = h_ref[...]
    gi = jnp.dot(x, wih_ref[...].T, preferred_element_type=jnp.float32) + bih_ref[...]
    gh = jnp.dot(h, whh_ref[...].T, preferred_element_type=jnp.float32) + bhh_ref[...]
    i_r, i_z, i_n = gi[:, :D], gi[:, D:2 * D], gi[:, 2 * D:]
    h_r, h_z, h_n = gh[:, :D], gh[:, D:2 * D], gh[:, 2 * D:]
    r = jax.nn.sigmoid(i_r + h_r)
    z = jax.nn.sigmoid(i_z + h_z)
    n = jnp.tanh(i_n + r * h_n)
    out_ref[...] = (1.0 - z) * n + z * h


def _gru_step(x, h, w_ih, w_hh, b_ih, b_hh):
    return pl.pallas_call(
        _gru_body,
        out_shape=jax.ShapeDtypeStruct((D, D), jnp.float32),
    )(x, h, w_ih, w_hh, b_ih.reshape(1, 3 * D), b_hh.reshape(1, 3 * D))


def _topk_pool(X, p, k):
    score = X @ p / jnp.linalg.norm(p)
    vals, idx = jax.lax.top_k(score, k)
    return X[idx] * jnp.tanh(vals)[:, None]


def kernel(x_t0, x_t1, edge_index_t0, edge_index_t1, W_init, p, w_ih, w_hh, b_ih, b_hh):
    pad = ((0, M - N), (0, 0))
    xs = [jnp.pad(x_t0, pad), jnp.pad(x_t1, pad)]
    padv = (N + jnp.arange(EPAD, dtype=jnp.int32) % (M - N))

    def _blockify(v):
        return jnp.concatenate([v, padv])

    srcs = [_blockify(edge_index_t0[0]), _blockify(edge_index_t1[0])]
    dsts = [_blockify(edge_index_t0[1]), _blockify(edge_index_t1[1])]

    degacc = _deg_sc(edge_index_t0[1], edge_index_t1[1])
    dinv = _dinv_tc(degacc)

    Ws = [W_init[l] for l in range(L)]
    out = None
    for t in range(T):
        h = xs[t]
        for l in range(L):
            X_tilde = _topk_pool(h[:N], p[l], D)
            Ws[l] = _gru_step(X_tilde, Ws[l], w_ih[l], w_hh[l], b_ih[l], b_hh[l])
            Y = (h @ Ws[l]) * dinv[t]
            acc = _conv_sc(Y, srcs[t], dsts[t])
            h = dinv[t] * (acc[0] + acc[1] - Y)
        out = h
    return out[:N]


# speedup vs baseline: 1.2662x; 1.2662x over previous
import functools

import jax
import jax.numpy as jnp
from jax import lax
from jax.experimental import pallas as pl
from jax.experimental.pallas import tpu as pltpu
from jax.experimental.pallas import tpu_sc as plsc

N = 10000
E = 320000
D = 128
L = 2
T = 2
M = 10240
RPS = M // 16
BLK = 128
NBLK = E // BLK
NW = 32
QB = 80
NBLKP = NW * QB
EPAD = NBLKP * BLK - E
INNER = 10

_mesh = plsc.VectorSubcoreMesh(core_axis_name="c", subcore_axis_name="s")


def _conv_body(y_hbm, src_hbm, dst_hbm, out_hbm, acc, sidx_all, didx_all,
               dblk, rows_a, rows_b, sem_a, sem_b, sem_i):
    c = lax.axis_index("c")
    s = lax.axis_index("s")
    wid = c * 16 + s
    base_e = wid * QB * BLK

    nch = RPS // BLK
    for k in range(nch):
        r0 = s * RPS + k * BLK
        pltpu.sync_copy(y_hbm.at[pl.ds(r0, BLK)], rows_a)
        pltpu.sync_copy(rows_a, acc.at[pl.ds(r0, BLK)])
    plsc.subcore_barrier()

    HQ = QB // 2
    HE = HQ * BLK

    def blk(b, carry):
        for i in range(BLK // 16):
            dblk[pl.ds(i * 16, 16)] = didx_all[pl.ds(b * BLK + i * 16, 16)]
        g = pltpu.async_copy(
            y_hbm.at[sidx_all.at[pl.ds(b * BLK, BLK)]], rows_a, sem_a)
        g.wait()
        pltpu.sync_copy(rows_a, acc.at[dblk], add=True)
        return carry

    for half in range(2):
        pltpu.sync_copy(src_hbm.at[pl.ds(base_e + half * HE, HE)], sidx_all)
        pltpu.sync_copy(dst_hbm.at[pl.ds(base_e + half * HE, HE)], didx_all)
        lax.fori_loop(0, HQ, blk, 0)
    plsc.subcore_barrier()

    for k in range(nch):
        r0 = s * RPS + k * BLK
        pltpu.sync_copy(acc.at[pl.ds(r0, BLK)], rows_a)
        pltpu.sync_copy(rows_a, out_hbm.at[c, pl.ds(r0, BLK)])


@jax.jit
def _conv_sc(y, src_blk, dst_blk):
    return pl.kernel(
        _conv_body,
        out_type=jax.ShapeDtypeStruct((2, M, D), jnp.float32),
        mesh=_mesh,
        scratch_types=[
            pltpu.VMEM_SHARED((M, D), jnp.float32),
            pltpu.VMEM((QB // 2 * BLK,), jnp.int32),
            pltpu.VMEM((QB // 2 * BLK,), jnp.int32),
            pltpu.VMEM((BLK,), jnp.int32),
            pltpu.VMEM((BLK, D), jnp.float32),
            pltpu.VMEM((BLK, D), jnp.float32),
            pltpu.SemaphoreType.DMA,
            pltpu.SemaphoreType.DMA,
            pltpu.SemaphoreType.DMA,
        ],
    )(y, src_blk, dst_blk)


def _deg_body(dst0_hbm, dst1_hbm, out_hbm, acc0, acc1, ones_v, didx, iov):
    c = lax.axis_index("c")
    s = lax.axis_index("s")
    wid = c * 16 + s

    for i in range(BLK):
        ones_v[i] = jnp.ones((16,), jnp.float32)
        iov[i] = jnp.zeros((16,), jnp.float32)
    for k in range(RPS // BLK):
        r0 = s * RPS + k * BLK
        pltpu.sync_copy(iov, acc0.at[pl.ds(r0, BLK)])
        pltpu.sync_copy(iov, acc1.at[pl.ds(r0, BLK)])
    plsc.subcore_barrier()

    q, r = divmod(NBLK, NW)
    nblk = q + jnp.where(wid < r, 1, 0)
    base = wid * q + jnp.minimum(wid, r)

    def blk0(j, carry):
        off = (base + j) * BLK
        pltpu.sync_copy(dst0_hbm.at[pl.ds(off, BLK)], didx)
        pltpu.sync_copy(ones_v, acc0.at[didx], add=True)
        return carry

    def blk1(j, carry):
        off = (base + j) * BLK
        pltpu.sync_copy(dst1_hbm.at[pl.ds(off, BLK)], didx)
        pltpu.sync_copy(ones_v, acc1.at[didx], add=True)
        return carry

    lax.fori_loop(0, nblk, blk0, 0)
    lax.fori_loop(0, nblk, blk1, 0)
    plsc.subcore_barrier()

    for k in range(RPS // BLK):
        r0 = s * RPS + k * BLK
        pltpu.sync_copy(acc0.at[pl.ds(r0, BLK)], iov)
        pltpu.sync_copy(iov, out_hbm.at[c, 0, pl.ds(r0, BLK)])
        pltpu.sync_copy(acc1.at[pl.ds(r0, BLK)], iov)
        pltpu.sync_copy(iov, out_hbm.at[c, 1, pl.ds(r0, BLK)])


@jax.jit
def _deg_sc(dst0, dst1):
    return pl.kernel(
        _deg_body,
        out_type=jax.ShapeDtypeStruct((2, 2, M, 16), jnp.float32),
        mesh=_mesh,
        scratch_types=[
            pltpu.VMEM_SHARED((M, 16), jnp.float32),
            pltpu.VMEM_SHARED((M, 16), jnp.float32),
            pltpu.VMEM((BLK, 16), jnp.float32),
            pltpu.VMEM((BLK,), jnp.int32),
            pltpu.VMEM((BLK, 16), jnp.float32),
        ],
    )(dst0, dst1)


def _dinv_body(da_ref, out_ref):
    d0 = da_ref[0, 0] + da_ref[1, 0]
    out_ref[0] = lax.rsqrt(jnp.sum(d0, axis=1, keepdims=True) + 1.0)
    d1 = da_ref[0, 1] + da_ref[1, 1]
    out_ref[1] = lax.rsqrt(jnp.sum(d1, axis=1, keepdims=True) + 1.0)


@jax.jit
def _dinv_tc(degacc):
    return pl.pallas_call(
        _dinv_body,
        out_shape=jax.ShapeDtypeStruct((2, M, 1), jnp.float32),
    )(degacc)


def _gru_body(x_ref, h_ref, wih_ref, whh_ref, bih_ref, bhh_ref, out_ref):
    x = x_ref[...]
    h = h_ref[...]
    gi = jnp.dot(x, wih_ref[...].T, preferred_element_type=jnp.float32) + bih_ref[...]
    gh = jnp.dot(h, whh_ref[...].T, preferred_element_type=jnp.float32) + bhh_ref[...]
    i_r, i_z, i_n = gi[:, :D], gi[:, D:2 * D], gi[:, 2 * D:]
    h_r, h_z, h_n = gh[:, :D], gh[:, D:2 * D], gh[:, 2 * D:]
    r = jax.nn.sigmoid(i_r + h_r)
    z = jax.nn.sigmoid(i_z + h_z)
    n = jnp.tanh(i_n + r * h_n)
    out_ref[...] = (1.0 - z) * n + z * h


def _gru_step(x, h, w_ih, w_hh, b_ih, b_hh):
    return pl.pallas_call(
        _gru_body,
        out_shape=jax.ShapeDtypeStruct((D, D), jnp.float32),
    )(x, h, w_ih, w_hh, b_ih.reshape(1, 3 * D), b_hh.reshape(1, 3 * D))


def _topk_pool(X, p, k):
    score = X @ p / jnp.linalg.norm(p)
    vals, idx = jax.lax.top_k(score, k)
    return X[idx] * jnp.tanh(vals)[:, None]


def kernel(x_t0, x_t1, edge_index_t0, edge_index_t1, W_init, p, w_ih, w_hh, b_ih, b_hh):
    pad = ((0, M - N), (0, 0))
    xs = [jnp.pad(x_t0, pad), jnp.pad(x_t1, pad)]
    padv = (N + jnp.arange(EPAD, dtype=jnp.int32) % (M - N))

    def _blockify(v):
        return jnp.concatenate([v, padv])

    srcs = [_blockify(edge_index_t0[0]), _blockify(edge_index_t1[0])]
    dsts = [_blockify(edge_index_t0[1]), _blockify(edge_index_t1[1])]

    degacc = _deg_sc(edge_index_t0[1], edge_index_t1[1])
    dinv = _dinv_tc(degacc)

    Ws = [W_init[l] for l in range(L)]
    out = None
    for t in range(T):
        h = xs[t]
        for l in range(L):
            X_tilde = _topk_pool(h[:N], p[l], D)
            Ws[l] = _gru_step(X_tilde, Ws[l], w_ih[l], w_hh[l], b_ih[l], b_hh[l])
            Y = (h @ Ws[l]) * dinv[t]
            acc = _conv_sc(Y, srcs[t], dsts[t])
            h = dinv[t] * (acc[0] + acc[1] - Y)
        out = h
    return out[:N]
